# baseline (speedup 1.0000x reference)
import jax
import jax.numpy as jnp
from jax.experimental import pallas as pl

B = 128
T = 577
VS = 768
TS = 768
D = 512


def _fused_head_kernel(vis_ref, txt_ref, wv_ref, bv_ref, wt_ref, bt_ref,
                       out_v_ref, out_t_ref):
    out_v_ref[...] = (
        jnp.dot(vis_ref[...], wv_ref[...], preferred_element_type=jnp.float32)
        + bv_ref[...]
    )
    out_t_ref[...] = (
        jnp.dot(txt_ref[...], wt_ref[...], preferred_element_type=jnp.float32)
        + bt_ref[...]
    )


def kernel(visual_feature, textual_feature, attribute_feature, att_nums,
           captions, Wv, bv, Wt, bt, Wp, bp, Wa, ba):
    del attribute_feature, att_nums, captions, Wp, bp, Wa, ba
    bv2 = bv.reshape(1, D)
    bt2 = bt.reshape(1, D)
    vis_flat = visual_feature.reshape(B, T * VS)
    out_v, out_t = pl.pallas_call(
        _fused_head_kernel,
        grid=(1,),
        in_specs=[
            pl.BlockSpec((B, VS), lambda i: (0, 0)),
            pl.BlockSpec((B, TS), lambda i: (0, 0)),
            pl.BlockSpec((VS, D), lambda i: (0, 0)),
            pl.BlockSpec((1, D), lambda i: (0, 0)),
            pl.BlockSpec((TS, D), lambda i: (0, 0)),
            pl.BlockSpec((1, D), lambda i: (0, 0)),
        ],
        out_specs=[
            pl.BlockSpec((B, D), lambda i: (0, 0)),
            pl.BlockSpec((B, D), lambda i: (0, 0)),
        ],
        out_shape=[
            jax.ShapeDtypeStruct((B, D), jnp.float32),
            jax.ShapeDtypeStruct((B, D), jnp.float32),
        ],
    )(vis_flat, textual_feature, Wv, bv2, Wt, bt2)
    return (out_v, out_t)


# speedup vs baseline: 1.9399x; 1.9399x over previous
import jax
import jax.numpy as jnp
from jax.experimental import pallas as pl

B = 128
T = 577
VS = 768
TS = 768
D = 512


def _fused_head_kernel(vis_ref, txt_ref, wv_ref, bv_ref, wt_ref, bt_ref,
                       out_v_ref, out_t_ref):
    out_v_ref[...] = (
        jnp.dot(vis_ref[:, 0, :], wv_ref[...],
                preferred_element_type=jnp.float32)
        + bv_ref[...]
    )
    out_t_ref[...] = (
        jnp.dot(txt_ref[...], wt_ref[...], preferred_element_type=jnp.float32)
        + bt_ref[...]
    )


def kernel(visual_feature, textual_feature, attribute_feature, att_nums,
           captions, Wv, bv, Wt, bt, Wp, bp, Wa, ba):
    del attribute_feature, att_nums, captions, Wp, bp, Wa, ba
    bv2 = bv.reshape(1, D)
    bt2 = bt.reshape(1, D)
    out_v, out_t = pl.pallas_call(
        _fused_head_kernel,
        grid=(1,),
        in_specs=[
            pl.BlockSpec((B, 8, VS), lambda i: (0, 0, 0)),
            pl.BlockSpec((B, TS), lambda i: (0, 0)),
            pl.BlockSpec((VS, D), lambda i: (0, 0)),
            pl.BlockSpec((1, D), lambda i: (0, 0)),
            pl.BlockSpec((TS, D), lambda i: (0, 0)),
            pl.BlockSpec((1, D), lambda i: (0, 0)),
        ],
        out_specs=[
            pl.BlockSpec((B, D), lambda i: (0, 0)),
            pl.BlockSpec((B, D), lambda i: (0, 0)),
        ],
        out_shape=[
            jax.ShapeDtypeStruct((B, D), jnp.float32),
            jax.ShapeDtypeStruct((B, D), jnp.float32),
        ],
    )(visual_feature, textual_feature, Wv, bv2, Wt, bt2)
    return (out_v, out_t)


# speedup vs baseline: 65.2699x; 33.6453x over previous
import jax
import jax.numpy as jnp
from jax.experimental import pallas as pl

B = 128
T = 577
VS = 768
TS = 768
D = 512


def _fused_head_kernel(vis_ref, txt_ref, wv_ref, bv_ref, wt_ref, bt_ref,
                       out_v_ref, out_t_ref):
    out_v_ref[...] = (
        jnp.dot(vis_ref[...], wv_ref[...],
                preferred_element_type=jnp.float32)
        + bv_ref[...]
    )
    out_t_ref[...] = (
        jnp.dot(txt_ref[...], wt_ref[...], preferred_element_type=jnp.float32)
        + bt_ref[...]
    )


def kernel(visual_feature, textual_feature, attribute_feature, att_nums,
           captions, Wv, bv, Wt, bt, Wp, bp, Wa, ba):
    del attribute_feature, att_nums, captions, Wp, bp, Wa, ba
    bv2 = bv.reshape(1, D)
    bt2 = bt.reshape(1, D)
    cls_tok = visual_feature[:, 0]
    out_v, out_t = pl.pallas_call(
        _fused_head_kernel,
        grid=(1,),
        in_specs=[
            pl.BlockSpec((B, VS), lambda i: (0, 0)),
            pl.BlockSpec((B, TS), lambda i: (0, 0)),
            pl.BlockSpec((VS, D), lambda i: (0, 0)),
            pl.BlockSpec((1, D), lambda i: (0, 0)),
            pl.BlockSpec((TS, D), lambda i: (0, 0)),
            pl.BlockSpec((1, D), lambda i: (0, 0)),
        ],
        out_specs=[
            pl.BlockSpec((B, D), lambda i: (0, 0)),
            pl.BlockSpec((B, D), lambda i: (0, 0)),
        ],
        out_shape=[
            jax.ShapeDtypeStruct((B, D), jnp.float32),
            jax.ShapeDtypeStruct((B, D), jnp.float32),
        ],
    )(cls_tok, textual_feature, Wv, bv2, Wt, bt2)
    return (out_v, out_t)
